# baseline (speedup 1.0000x reference)
import jax
import jax.numpy as jnp
from jax.experimental import pallas as pl
from jax.experimental.pallas import tpu as pltpu


def _roll_body(x_ref, buf_ref, out_ref, sem_shift, sem_last):
    m = buf_ref.shape[0]
    shift = pltpu.make_async_copy(
        buf_ref.at[pl.ds(1, m - 1)], out_ref.at[pl.ds(0, m - 1)], sem_shift
    )
    last = pltpu.make_async_copy(x_ref, out_ref.at[pl.ds(m - 1, 1)], sem_last)
    shift.start()
    last.start()
    shift.wait()
    last.wait()


def kernel(input, buffer):
    m, b, d = buffer.shape
    x = input.reshape(1, b, d)
    return pl.pallas_call(
        _roll_body,
        in_specs=[
            pl.BlockSpec(memory_space=pl.ANY),
            pl.BlockSpec(memory_space=pl.ANY),
        ],
        out_specs=pl.BlockSpec(memory_space=pl.ANY),
        out_shape=jax.ShapeDtypeStruct((m, b, d), buffer.dtype),
        scratch_shapes=[pltpu.SemaphoreType.DMA, pltpu.SemaphoreType.DMA],
    )(x, buffer)
